# baseline (speedup 1.0000x reference)
import functools

import jax
import jax.numpy as jnp
from jax import lax
from jax.experimental import pallas as pl
from jax.experimental.pallas import tpu as pltpu
from jax.experimental.pallas import tpu_sc as plsc

B, QH, QW = 2, 100, 100
D, H, K, SCALES = 256, 8, 8, 1
DK = D // H
DFF = 1024
P = QH * QW
Q = B * P
NITEMS = Q * H
NE = NITEMS * 32

_QB_A = 1000


NB = Q // 8


def _to_phys(x):
    r = x.shape[0]
    return x.reshape(r // 8, 8, 2, 128).transpose(0, 2, 1, 3)


def _from_phys(x):
    r = x.shape[0] * 8
    return x.transpose(0, 2, 1, 3).reshape(r, 256)


def _prep_body(srcq_ref, src_ref, Wb_ref, bb_ref, Wcat_ref, bcat_ref,
               Wk_ref, bk_ref, off_ref, alog_ref, tbl_ref):
    sq = jnp.dot(srcq_ref[...], Wb_ref[...], preferred_element_type=jnp.float32) + bb_ref[...]
    offa = jnp.dot(sq, Wcat_ref[...], preferred_element_type=jnp.float32) + bcat_ref[...]
    off_ref[...] = offa[:, :2 * H * K]
    alog_ref[...] = offa[:, 2 * H * K:]
    tbl = jnp.dot(src_ref[...], Wk_ref[...], preferred_element_type=jnp.float32) + bk_ref[...]
    tbl_ref[...] = _to_phys(tbl)


def _run_prep(srcq2, src2, Wb, bb, Wcat, bcat, Wk, bk):
    n = 2 * H * K + H * K
    return pl.pallas_call(
        _prep_body,
        grid=(Q // _QB_A,),
        in_specs=[
            pl.BlockSpec((_QB_A, D), lambda i: (i, 0)),
            pl.BlockSpec((_QB_A, D), lambda i: (i, 0)),
            pl.BlockSpec((D, D), lambda i: (0, 0)),
            pl.BlockSpec((1, D), lambda i: (0, 0)),
            pl.BlockSpec((D, n), lambda i: (0, 0)),
            pl.BlockSpec((1, n), lambda i: (0, 0)),
            pl.BlockSpec((D, D), lambda i: (0, 0)),
            pl.BlockSpec((1, D), lambda i: (0, 0)),
        ],
        out_specs=[
            pl.BlockSpec((_QB_A, 2 * H * K), lambda i: (i, 0)),
            pl.BlockSpec((_QB_A, H * K), lambda i: (i, 0)),
            pl.BlockSpec((_QB_A // 8, 2, 8, 128), lambda i: (i, 0, 0, 0)),
        ],
        out_shape=[
            jax.ShapeDtypeStruct((Q, 2 * H * K), jnp.float32),
            jax.ShapeDtypeStruct((Q, H * K), jnp.float32),
            jax.ShapeDtypeStruct((NB, 2, 8, 128), jnp.float32),
        ],
    )(srcq2, src2, Wb, bb, Wcat, bcat, Wk, bk)


_QB_B = 2000

import numpy as _np

_LANE = _np.arange(256)
_LH = _LANE // 32
_LC = (_LANE % 32) // 8
_I64 = _np.arange(64)
_REP = (_I64[:, None] == (_LH * 8 + _LANE % 8)[None, :]).astype(_np.float32)
_RSUM = ((_I64 // 8)[:, None] == _LH[None, :]).astype(_np.float32)
_RX = _np.concatenate([_REP * (_LC % 2 == 0)[None, :], _REP * (_LC % 2 == 1)[None, :]], axis=0)
_RY = _np.concatenate([_REP * (_LC // 2 == 0)[None, :], _REP * (_LC // 2 == 1)[None, :]], axis=0)
_T2 = (_np.arange(2)[:, None] == ((_I64 // 8) % 2)[None, :]).astype(_np.float32)
_ROWC = ((_LH // 4) * 32 + _LH % 4).astype(_np.float32)


def _addr_body(off_ref, alog_ref, rp0_ref, rp1_ref, rep_ref, rsum_ref,
               rx_ref, ry_ref, t2_ref, idx_ref, w_ref):
    b = pl.program_id(0)
    off = off_ref[...]
    alog = alog_ref[...]
    rp0 = rp0_ref[0]
    rp1 = rp1_ref[0]
    offx = off[:, :64]
    offy = off[:, 64:]
    rpx2 = jnp.concatenate([rp0[:, 0:1], rp1[:, 0:1]], axis=1) * float(QW - 1)
    rpy2 = jnp.concatenate([rp0[:, 1:2], rp1[:, 1:2]], axis=1) * float(QH - 1)
    t2 = t2_ref[...]
    rrpx = jnp.dot(rpx2, t2, preferred_element_type=jnp.float32, precision=lax.Precision.HIGHEST)
    rrpy = jnp.dot(rpy2, t2, preferred_element_type=jnp.float32, precision=lax.Precision.HIGHEST)
    ptx = rrpx + offx
    pty = rrpy + offy
    vx = 2.0 * ptx / float(QW - 1) - 1.0
    vy = 2.0 * pty / float(QH - 1) - 1.0
    sx = ((vx + 1.0) * float(QW) - 1.0) / 2.0
    sy = ((vy + 1.0) * float(QH) - 1.0) / 2.0
    x0 = jnp.floor(sx)
    y0 = jnp.floor(sy)
    x1 = x0 + 1.0
    y1 = y0 + 1.0
    wx1 = sx - x0
    wx0 = 1.0 - wx1
    wy1 = sy - y0
    wy0 = 1.0 - wy1

    def fvalid(cf, lim):
        return ((cf >= 0.0) & (cf <= lim)).astype(jnp.float32)

    wvx = jnp.concatenate([wx0 * fvalid(x0, float(QW - 1)),
                           wx1 * fvalid(x1, float(QW - 1))], axis=1)
    wvy = jnp.concatenate([wy0 * fvalid(y0, float(QH - 1)),
                           wy1 * fvalid(y1, float(QH - 1))], axis=1)
    xi = jnp.concatenate([jnp.clip(x0, 0.0, float(QW - 1)),
                          jnp.clip(x1, 0.0, float(QW - 1))], axis=1)
    yi = jnp.concatenate([jnp.clip(y0, 0.0, float(QH - 1)),
                          jnp.clip(y1, 0.0, float(QH - 1))], axis=1)
    rx = rx_ref[...]
    ry = ry_ref[...]
    wvx256 = jnp.dot(wvx, rx, preferred_element_type=jnp.float32, precision=lax.Precision.HIGHEST)
    wvy256 = jnp.dot(wvy, ry, preferred_element_type=jnp.float32, precision=lax.Precision.HIGHEST)
    xi256 = jnp.dot(xi, rx, preferred_element_type=jnp.float32, precision=lax.Precision.HIGHEST)
    yi256 = jnp.dot(yi, ry, preferred_element_type=jnp.float32, precision=lax.Precision.HIGHEST)
    ea = jnp.exp(alog)
    a256 = jnp.dot(ea, rep_ref[...], preferred_element_type=jnp.float32, precision=lax.Precision.HIGHEST)
    s256 = jnp.dot(ea, rsum_ref[...], preferred_element_type=jnp.float32, precision=lax.Precision.HIGHEST)
    w256 = a256 / s256 * wvx256 * wvy256
    qs = (b * P).astype(jnp.float32) + yi256 * float(QW) + xi256
    qs8 = jnp.floor(qs * 0.125)
    hh = lax.broadcasted_iota(jnp.int32, (1, 256), 1) // 32
    rowc = ((hh // 4) * 32 + hh % 4).astype(jnp.float32)
    row = qs8 * 64.0 + (qs - qs8 * 8.0) * 4.0 + rowc
    idx_ref[...] = _to_phys(row.astype(jnp.int32))
    w_ref[...] = _to_phys(w256)


def _run_addr(off, alog, rp3):
    qbb = _QB_B
    return pl.pallas_call(
        _addr_body,
        grid=(B, P // qbb),
        in_specs=[
            pl.BlockSpec((qbb, 2 * H * K), lambda b, p: (b * (P // qbb) + p, 0)),
            pl.BlockSpec((qbb, H * K), lambda b, p: (b * (P // qbb) + p, 0)),
            pl.BlockSpec((1, qbb, 2), lambda b, p: (0, p, 0)),
            pl.BlockSpec((1, qbb, 2), lambda b, p: (1, p, 0)),
            pl.BlockSpec((64, 256), lambda b, p: (0, 0)),
            pl.BlockSpec((64, 256), lambda b, p: (0, 0)),
            pl.BlockSpec((128, 256), lambda b, p: (0, 0)),
            pl.BlockSpec((128, 256), lambda b, p: (0, 0)),
            pl.BlockSpec((2, 64), lambda b, p: (0, 0)),
        ],
        out_specs=[
            pl.BlockSpec((qbb // 8, 2, 8, 128), lambda b, p: (b * (P // qbb) + p, 0, 0, 0)),
            pl.BlockSpec((qbb // 8, 2, 8, 128), lambda b, p: (b * (P // qbb) + p, 0, 0, 0)),
        ],
        out_shape=[
            jax.ShapeDtypeStruct((NB, 2, 8, 128), jnp.int32),
            jax.ShapeDtypeStruct((NB, 2, 8, 128), jnp.float32),
        ],
    )(off, alog, rp3, rp3, _REP, _RSUM, _RX, _RY, _T2)


_NW = 32
_IPW = NITEMS // _NW
_CHUNK = 20
_NCHUNK = _IPW // _CHUNK
_CE = _CHUNK * 32
_GPC = _CE // 128


def _sc_body(idx_hbm, w_hbm, tbl_hbm, out_hbm, idx_v, w_v, rows_v, out_v,
             isem0, isem1, gsem0, gsem1, osem0, osem1):
    wid = lax.axis_index("s") * 2 + lax.axis_index("c")
    base_item = wid * _IPW
    isem = (isem0, isem1)
    gsem = (gsem0, gsem1)
    osem = (osem0, osem1)

    def e_of(c):
        return (base_item + c * _CHUNK) * 32

    def issue_idxw(c, q):
        e0 = e_of(c)
        pltpu.async_copy(idx_hbm.at[pl.ds(e0, _CE)], idx_v.at[q], isem[q])
        pltpu.async_copy(w_hbm.at[pl.ds(e0, _CE)], w_v.at[q], isem[q])

    def wait_idxw(q):
        pltpu.make_async_copy(idx_hbm.at[pl.ds(0, _CE)], idx_v.at[q], isem[q]).wait()
        pltpu.make_async_copy(w_hbm.at[pl.ds(0, _CE)], w_v.at[q], isem[q]).wait()

    def issue_gathers(q):
        for g in range(_GPC):
            pltpu.async_copy(tbl_hbm.at[idx_v.at[q, pl.ds(g * 128, 128)]],
                             rows_v.at[q, pl.ds(g * 128, 128)], gsem[q])

    def drain_gathers(q):
        pltpu.make_async_copy(tbl_hbm.at[pl.ds(0, _CE)], rows_v.at[q], gsem[q]).wait()

    def wait_out(q):
        pltpu.make_async_copy(out_v.at[q], out_hbm.at[pl.ds(0, _CE)], osem[q]).wait()

    def compute(c, q):
        def item_body(i, carry2):
            rbase = i * 32
            wvA = w_v[q, pl.ds(rbase, 16)]
            wvB = w_v[q, pl.ds(rbase + 16, 16)]
            a0 = jnp.zeros((16,), jnp.float32)
            a1 = jnp.zeros((16,), jnp.float32)
            a2 = jnp.zeros((16,), jnp.float32)
            a3 = jnp.zeros((16,), jnp.float32)
            for r in range(16):
                wgA = wvA[r]
                wgB = wvB[r]
                a0 = a0 + wgA * rows_v[q, rbase + r, pl.ds(0, 16)]
                a1 = a1 + wgA * rows_v[q, rbase + r, pl.ds(16, 16)]
                a2 = a2 + wgB * rows_v[q, rbase + 16 + r, pl.ds(0, 16)]
                a3 = a3 + wgB * rows_v[q, rbase + 16 + r, pl.ds(16, 16)]
            out_v[q, pl.ds(rbase, 16)] = a0 + a2
            out_v[q, pl.ds(rbase + 16, 16)] = a1 + a3
            return carry2

        lax.fori_loop(0, _CHUNK, item_body, 0)
        pltpu.async_copy(out_v.at[q], out_hbm.at[pl.ds(e_of(c), _CE)], osem[q])

    def step(c, q):
        @pl.when(c < _NCHUNK - 1)
        def _():
            wait_idxw(1 - q)
            issue_gathers(1 - q)
        drain_gathers(q)

        @pl.when(c >= 2)
        def _():
            wait_out(q)
        compute(c, q)

        @pl.when(c + 2 <= _NCHUNK - 1)
        def _():
            issue_idxw(c + 2, q)

    pltpu.sync_copy(idx_hbm.at[pl.ds(e_of(0), _CE)], idx_v.at[0])
    pltpu.sync_copy(w_hbm.at[pl.ds(e_of(0), _CE)], w_v.at[0])
    issue_gathers(0)
    issue_idxw(1, 1)

    def pair_body(j, carry):
        step(2 * j, 0)
        step(2 * j + 1, 1)
        return carry

    lax.fori_loop(0, _NCHUNK // 2, pair_body, 0)
    wait_out(0)
    wait_out(1)


def _run_sc(idx2, wflat, tbl):
    mesh = plsc.VectorSubcoreMesh(core_axis_name="c", subcore_axis_name="s")
    f = pl.kernel(
        _sc_body,
        out_type=jax.ShapeDtypeStruct((NE,), jnp.float32),
        mesh=mesh,
        compiler_params=pltpu.CompilerParams(use_tc_tiling_on_sc=False),
        scratch_types=[
            pltpu.VMEM((2, _CE), jnp.int32),
            pltpu.VMEM((2, _CE), jnp.float32),
            pltpu.VMEM((2, _CE, 32), jnp.float32),
            pltpu.VMEM((2, _CE), jnp.float32),
            pltpu.SemaphoreType.DMA,
            pltpu.SemaphoreType.DMA,
            pltpu.SemaphoreType.DMA,
            pltpu.SemaphoreType.DMA,
            pltpu.SemaphoreType.DMA,
            pltpu.SemaphoreType.DMA,
        ],
    )
    return f(idx2, wflat, tbl)


_QB_D = 1000


def _post_body(feat_ref, Wm_ref, bm_ref, W1_ref, b1_ref, W2_ref, b2_ref,
               g_ref, be_ref, out_ref):
    feat = _from_phys(feat_ref[...])
    x = jnp.dot(feat, Wm_ref[...], preferred_element_type=jnp.float32) + bm_ref[...]
    hh = jnp.maximum(jnp.dot(x, W1_ref[...], preferred_element_type=jnp.float32) + b1_ref[...], 0.0)
    y = jnp.dot(hh, W2_ref[...], preferred_element_type=jnp.float32) + b2_ref[...] + x
    mu = jnp.mean(y, axis=-1, keepdims=True)
    var = jnp.mean((y - mu) ** 2, axis=-1, keepdims=True)
    out_ref[...] = (y - mu) / jnp.sqrt(var + 1e-5) * g_ref[...] + be_ref[...]


def _run_post(feat2, Wm, bm, W1, b1, W2, b2, gamma, beta):
    return pl.pallas_call(
        _post_body,
        grid=(Q // _QB_D,),
        in_specs=[
            pl.BlockSpec((_QB_D // 8, 2, 8, 128), lambda i: (i, 0, 0, 0)),
            pl.BlockSpec((D, D), lambda i: (0, 0)),
            pl.BlockSpec((1, D), lambda i: (0, 0)),
            pl.BlockSpec((D, DFF), lambda i: (0, 0)),
            pl.BlockSpec((1, DFF), lambda i: (0, 0)),
            pl.BlockSpec((DFF, D), lambda i: (0, 0)),
            pl.BlockSpec((1, D), lambda i: (0, 0)),
            pl.BlockSpec((1, D), lambda i: (0, 0)),
            pl.BlockSpec((1, D), lambda i: (0, 0)),
        ],
        out_specs=[pl.BlockSpec((_QB_D, D), lambda i: (i, 0))],
        out_shape=[jax.ShapeDtypeStruct((Q, D), jnp.float32)],
    )(feat2, Wm, bm, W1, b1, W2, b2, gamma, beta)[0]


def kernel(src, ref_point, src_query, Wq, bq, Wb, bb, Wk, bk, Woff, boff,
           Wa, ba, Wm, bm, W1, b1, W2, b2, gamma, beta):
    del Wq, bq
    srcq2 = src_query.reshape(Q, D)
    src2 = src.reshape(Q, D)
    rp3 = ref_point.reshape(B, P, 2)
    Woff2 = Woff.reshape(D, H, K, 2).transpose(0, 3, 1, 2).reshape(D, 2 * H * K)
    boff2 = boff.reshape(H, K, 2).transpose(2, 0, 1).reshape(2 * H * K)
    Wcat = jnp.concatenate([Woff2, Wa], axis=1)
    bcat = jnp.concatenate([boff2, ba]).reshape(1, -1)

    off, alog, tbl_t = _run_prep(srcq2, src2, Wb, bb.reshape(1, D), Wcat, bcat,
                                 Wk, bk.reshape(1, D))
    idx_t, w_t = _run_addr(off, alog, rp3)

    featflat = _run_sc(idx_t.reshape(NE), w_t.reshape(NE),
                       tbl_t.reshape(NITEMS, DK))

    featp = featflat.reshape(NB, 2, 8, 128)
    out = _run_post(featp, Wm, bm.reshape(1, D), W1, b1.reshape(1, DFF),
                    W2, b2.reshape(1, D), gamma.reshape(1, D), beta.reshape(1, D))
    return out.reshape(B, QH, QW, D)


# speedup vs baseline: 1.0875x; 1.0875x over previous
import functools

import jax
import jax.numpy as jnp
from jax import lax
from jax.experimental import pallas as pl
from jax.experimental.pallas import tpu as pltpu
from jax.experimental.pallas import tpu_sc as plsc

B, QH, QW = 2, 100, 100
D, H, K, SCALES = 256, 8, 8, 1
DK = D // H
DFF = 1024
P = QH * QW
Q = B * P
NITEMS = Q * H
NE = NITEMS * 32

_QB_A = 1000


NB = Q // 8


def _to_phys(x):
    r = x.shape[0]
    return x.reshape(r // 8, 8, 2, 128).transpose(0, 2, 1, 3)


def _from_phys(x):
    r = x.shape[0] * 8
    return x.transpose(0, 2, 1, 3).reshape(r, 256)


def _prep_body(srcq_ref, src_ref, Wb_ref, bb_ref, Wcat_ref, bcat_ref,
               Wk_ref, bk_ref, off_ref, alog_ref, tbl_ref):
    sq = jnp.dot(srcq_ref[...], Wb_ref[...], preferred_element_type=jnp.float32) + bb_ref[...]
    offa = jnp.dot(sq, Wcat_ref[...], preferred_element_type=jnp.float32) + bcat_ref[...]
    off_ref[...] = offa[:, :2 * H * K]
    alog_ref[...] = offa[:, 2 * H * K:]
    tbl = jnp.dot(src_ref[...], Wk_ref[...], preferred_element_type=jnp.float32) + bk_ref[...]
    tbl_ref[...] = _to_phys(tbl)


def _run_prep(srcq2, src2, Wb, bb, Wcat, bcat, Wk, bk):
    n = 2 * H * K + H * K
    return pl.pallas_call(
        _prep_body,
        grid=(Q // _QB_A,),
        in_specs=[
            pl.BlockSpec((_QB_A, D), lambda i: (i, 0)),
            pl.BlockSpec((_QB_A, D), lambda i: (i, 0)),
            pl.BlockSpec((D, D), lambda i: (0, 0)),
            pl.BlockSpec((1, D), lambda i: (0, 0)),
            pl.BlockSpec((D, n), lambda i: (0, 0)),
            pl.BlockSpec((1, n), lambda i: (0, 0)),
            pl.BlockSpec((D, D), lambda i: (0, 0)),
            pl.BlockSpec((1, D), lambda i: (0, 0)),
        ],
        out_specs=[
            pl.BlockSpec((_QB_A, 2 * H * K), lambda i: (i, 0)),
            pl.BlockSpec((_QB_A, H * K), lambda i: (i, 0)),
            pl.BlockSpec((_QB_A // 8, 2, 8, 128), lambda i: (i, 0, 0, 0)),
        ],
        out_shape=[
            jax.ShapeDtypeStruct((Q, 2 * H * K), jnp.float32),
            jax.ShapeDtypeStruct((Q, H * K), jnp.float32),
            jax.ShapeDtypeStruct((NB, 2, 8, 128), jnp.float32),
        ],
    )(srcq2, src2, Wb, bb, Wcat, bcat, Wk, bk)


_QB_B = 2000

import numpy as _np

_LANE = _np.arange(256)
_LH = _LANE // 32
_LC = (_LANE % 32) // 8
_I64 = _np.arange(64)
_REP = (_I64[:, None] == (_LH * 8 + _LANE % 8)[None, :]).astype(_np.float32)
_RSUM = ((_I64 // 8)[:, None] == _LH[None, :]).astype(_np.float32)
_RX = _np.concatenate([_REP * (_LC % 2 == 0)[None, :], _REP * (_LC % 2 == 1)[None, :]], axis=0)
_RY = _np.concatenate([_REP * (_LC // 2 == 0)[None, :], _REP * (_LC // 2 == 1)[None, :]], axis=0)
_T2 = (_np.arange(2)[:, None] == ((_I64 // 8) % 2)[None, :]).astype(_np.float32)
_ROWC = ((_LH // 4) * 32 + _LH % 4).astype(_np.float32)


def _addr_body(off_ref, alog_ref, rp0_ref, rp1_ref, rep_ref, rsum_ref,
               rx_ref, ry_ref, t2_ref, idx_ref, w_ref):
    b = pl.program_id(0)
    off = off_ref[...]
    alog = alog_ref[...]
    rp0 = rp0_ref[0]
    rp1 = rp1_ref[0]
    offx = off[:, :64]
    offy = off[:, 64:]
    rpx2 = jnp.concatenate([rp0[:, 0:1], rp1[:, 0:1]], axis=1) * float(QW - 1)
    rpy2 = jnp.concatenate([rp0[:, 1:2], rp1[:, 1:2]], axis=1) * float(QH - 1)
    t2 = t2_ref[...]
    rrpx = jnp.dot(rpx2, t2, preferred_element_type=jnp.float32, precision=lax.Precision.HIGHEST)
    rrpy = jnp.dot(rpy2, t2, preferred_element_type=jnp.float32, precision=lax.Precision.HIGHEST)
    ptx = rrpx + offx
    pty = rrpy + offy
    vx = 2.0 * ptx / float(QW - 1) - 1.0
    vy = 2.0 * pty / float(QH - 1) - 1.0
    sx = ((vx + 1.0) * float(QW) - 1.0) / 2.0
    sy = ((vy + 1.0) * float(QH) - 1.0) / 2.0
    x0 = jnp.floor(sx)
    y0 = jnp.floor(sy)
    x1 = x0 + 1.0
    y1 = y0 + 1.0
    wx1 = sx - x0
    wx0 = 1.0 - wx1
    wy1 = sy - y0
    wy0 = 1.0 - wy1

    def fvalid(cf, lim):
        return ((cf >= 0.0) & (cf <= lim)).astype(jnp.float32)

    wvx = jnp.concatenate([wx0 * fvalid(x0, float(QW - 1)),
                           wx1 * fvalid(x1, float(QW - 1))], axis=1)
    wvy = jnp.concatenate([wy0 * fvalid(y0, float(QH - 1)),
                           wy1 * fvalid(y1, float(QH - 1))], axis=1)
    xi = jnp.concatenate([jnp.clip(x0, 0.0, float(QW - 1)),
                          jnp.clip(x1, 0.0, float(QW - 1))], axis=1)
    yi = jnp.concatenate([jnp.clip(y0, 0.0, float(QH - 1)),
                          jnp.clip(y1, 0.0, float(QH - 1))], axis=1)
    rx = rx_ref[...]
    ry = ry_ref[...]
    wvx256 = jnp.dot(wvx, rx, preferred_element_type=jnp.float32, precision=lax.Precision.HIGHEST)
    wvy256 = jnp.dot(wvy, ry, preferred_element_type=jnp.float32, precision=lax.Precision.HIGHEST)
    xi256 = jnp.dot(xi, rx, preferred_element_type=jnp.float32, precision=lax.Precision.HIGHEST)
    yi256 = jnp.dot(yi, ry, preferred_element_type=jnp.float32, precision=lax.Precision.HIGHEST)
    ea = jnp.exp(alog)
    a256 = jnp.dot(ea, rep_ref[...], preferred_element_type=jnp.float32, precision=lax.Precision.HIGHEST)
    s256 = jnp.dot(ea, rsum_ref[...], preferred_element_type=jnp.float32, precision=lax.Precision.HIGHEST)
    w256 = a256 / s256 * wvx256 * wvy256
    qs = (b * P).astype(jnp.float32) + yi256 * float(QW) + xi256
    qs8 = jnp.floor(qs * 0.125)
    hh = lax.broadcasted_iota(jnp.int32, (1, 256), 1) // 32
    rowc = ((hh // 4) * 32 + hh % 4).astype(jnp.float32)
    row = qs8 * 64.0 + (qs - qs8 * 8.0) * 4.0 + rowc
    idx_ref[...] = _to_phys(row.astype(jnp.int32))
    w_ref[...] = _to_phys(w256)


def _run_addr(off, alog, rp3):
    qbb = _QB_B
    return pl.pallas_call(
        _addr_body,
        grid=(B, P // qbb),
        in_specs=[
            pl.BlockSpec((qbb, 2 * H * K), lambda b, p: (b * (P // qbb) + p, 0)),
            pl.BlockSpec((qbb, H * K), lambda b, p: (b * (P // qbb) + p, 0)),
            pl.BlockSpec((1, qbb, 2), lambda b, p: (0, p, 0)),
            pl.BlockSpec((1, qbb, 2), lambda b, p: (1, p, 0)),
            pl.BlockSpec((64, 256), lambda b, p: (0, 0)),
            pl.BlockSpec((64, 256), lambda b, p: (0, 0)),
            pl.BlockSpec((128, 256), lambda b, p: (0, 0)),
            pl.BlockSpec((128, 256), lambda b, p: (0, 0)),
            pl.BlockSpec((2, 64), lambda b, p: (0, 0)),
        ],
        out_specs=[
            pl.BlockSpec((qbb // 8, 2, 8, 128), lambda b, p: (b * (P // qbb) + p, 0, 0, 0)),
            pl.BlockSpec((qbb // 8, 2, 8, 128), lambda b, p: (b * (P // qbb) + p, 0, 0, 0)),
        ],
        out_shape=[
            jax.ShapeDtypeStruct((NB, 2, 8, 128), jnp.int32),
            jax.ShapeDtypeStruct((NB, 2, 8, 128), jnp.float32),
        ],
    )(off, alog, rp3, rp3, _REP, _RSUM, _RX, _RY, _T2)


_NW = 32
_IPW = NITEMS // _NW
_CHUNK = 40
_NCHUNK = _IPW // _CHUNK
_CE = _CHUNK * 32
_GPC = _CE // 128


def _sc_body(idx_hbm, w_hbm, tbl_hbm, out_hbm, idx_v, w_v, rows_v, out_v,
             isem0, isem1, gsem0, gsem1, osem0, osem1):
    wid = lax.axis_index("s") * 2 + lax.axis_index("c")
    base_item = wid * _IPW
    isem = (isem0, isem1)
    gsem = (gsem0, gsem1)
    osem = (osem0, osem1)

    def e_of(c):
        return (base_item + c * _CHUNK) * 32

    def issue_idxw(c, q):
        e0 = e_of(c)
        pltpu.async_copy(idx_hbm.at[pl.ds(e0, _CE)], idx_v.at[q], isem[q])
        pltpu.async_copy(w_hbm.at[pl.ds(e0, _CE)], w_v.at[q], isem[q])

    def wait_idxw(q):
        pltpu.make_async_copy(idx_hbm.at[pl.ds(0, _CE)], idx_v.at[q], isem[q]).wait()
        pltpu.make_async_copy(w_hbm.at[pl.ds(0, _CE)], w_v.at[q], isem[q]).wait()

    def issue_gathers(q):
        for g in range(_GPC):
            pltpu.async_copy(tbl_hbm.at[idx_v.at[q, pl.ds(g * 128, 128)]],
                             rows_v.at[q, pl.ds(g * 128, 128)], gsem[q])

    def drain_gathers(q):
        pltpu.make_async_copy(tbl_hbm.at[pl.ds(0, _CE)], rows_v.at[q], gsem[q]).wait()

    def wait_out(q):
        pltpu.make_async_copy(out_v.at[q], out_hbm.at[pl.ds(0, _CE)], osem[q]).wait()

    def compute(c, q):
        def item_body(i, carry2):
            rbase = i * 32
            wvA = w_v[q, pl.ds(rbase, 16)]
            wvB = w_v[q, pl.ds(rbase + 16, 16)]
            a0 = jnp.zeros((16,), jnp.float32)
            a1 = jnp.zeros((16,), jnp.float32)
            a2 = jnp.zeros((16,), jnp.float32)
            a3 = jnp.zeros((16,), jnp.float32)
            for r in range(16):
                wgA = wvA[r]
                wgB = wvB[r]
                a0 = a0 + wgA * rows_v[q, rbase + r, pl.ds(0, 16)]
                a1 = a1 + wgA * rows_v[q, rbase + r, pl.ds(16, 16)]
                a2 = a2 + wgB * rows_v[q, rbase + 16 + r, pl.ds(0, 16)]
                a3 = a3 + wgB * rows_v[q, rbase + 16 + r, pl.ds(16, 16)]
            out_v[q, pl.ds(rbase, 16)] = a0 + a2
            out_v[q, pl.ds(rbase + 16, 16)] = a1 + a3
            return carry2

        lax.fori_loop(0, _CHUNK, item_body, 0)
        pltpu.async_copy(out_v.at[q], out_hbm.at[pl.ds(e_of(c), _CE)], osem[q])

    def step(c, q):
        @pl.when(c < _NCHUNK - 1)
        def _():
            wait_idxw(1 - q)
            issue_gathers(1 - q)
        drain_gathers(q)

        @pl.when(c >= 2)
        def _():
            wait_out(q)
        compute(c, q)

        @pl.when(c + 2 <= _NCHUNK - 1)
        def _():
            issue_idxw(c + 2, q)

    pltpu.sync_copy(idx_hbm.at[pl.ds(e_of(0), _CE)], idx_v.at[0])
    pltpu.sync_copy(w_hbm.at[pl.ds(e_of(0), _CE)], w_v.at[0])
    issue_gathers(0)
    issue_idxw(1, 1)

    def pair_body(j, carry):
        step(2 * j, 0)
        step(2 * j + 1, 1)
        return carry

    lax.fori_loop(0, _NCHUNK // 2, pair_body, 0)
    if _NCHUNK % 2:
        step(_NCHUNK - 1, 0)
    wait_out(0)
    wait_out(1)


def _run_sc(idx2, wflat, tbl):
    mesh = plsc.VectorSubcoreMesh(core_axis_name="c", subcore_axis_name="s")
    f = pl.kernel(
        _sc_body,
        out_type=jax.ShapeDtypeStruct((NE,), jnp.float32),
        mesh=mesh,
        compiler_params=pltpu.CompilerParams(use_tc_tiling_on_sc=False),
        scratch_types=[
            pltpu.VMEM((2, _CE), jnp.int32),
            pltpu.VMEM((2, _CE), jnp.float32),
            pltpu.VMEM((2, _CE, 32), jnp.float32),
            pltpu.VMEM((2, _CE), jnp.float32),
            pltpu.SemaphoreType.DMA,
            pltpu.SemaphoreType.DMA,
            pltpu.SemaphoreType.DMA,
            pltpu.SemaphoreType.DMA,
            pltpu.SemaphoreType.DMA,
            pltpu.SemaphoreType.DMA,
        ],
    )
    return f(idx2, wflat, tbl)


_QB_D = 1000


def _post_body(feat_ref, Wm_ref, bm_ref, W1_ref, b1_ref, W2_ref, b2_ref,
               g_ref, be_ref, out_ref):
    feat = _from_phys(feat_ref[...])
    x = jnp.dot(feat, Wm_ref[...], preferred_element_type=jnp.float32) + bm_ref[...]
    hh = jnp.maximum(jnp.dot(x, W1_ref[...], preferred_element_type=jnp.float32) + b1_ref[...], 0.0)
    y = jnp.dot(hh, W2_ref[...], preferred_element_type=jnp.float32) + b2_ref[...] + x
    mu = jnp.mean(y, axis=-1, keepdims=True)
    var = jnp.mean((y - mu) ** 2, axis=-1, keepdims=True)
    out_ref[...] = (y - mu) / jnp.sqrt(var + 1e-5) * g_ref[...] + be_ref[...]


def _run_post(feat2, Wm, bm, W1, b1, W2, b2, gamma, beta):
    return pl.pallas_call(
        _post_body,
        grid=(Q // _QB_D,),
        in_specs=[
            pl.BlockSpec((_QB_D // 8, 2, 8, 128), lambda i: (i, 0, 0, 0)),
            pl.BlockSpec((D, D), lambda i: (0, 0)),
            pl.BlockSpec((1, D), lambda i: (0, 0)),
            pl.BlockSpec((D, DFF), lambda i: (0, 0)),
            pl.BlockSpec((1, DFF), lambda i: (0, 0)),
            pl.BlockSpec((DFF, D), lambda i: (0, 0)),
            pl.BlockSpec((1, D), lambda i: (0, 0)),
            pl.BlockSpec((1, D), lambda i: (0, 0)),
            pl.BlockSpec((1, D), lambda i: (0, 0)),
        ],
        out_specs=[pl.BlockSpec((_QB_D, D), lambda i: (i, 0))],
        out_shape=[jax.ShapeDtypeStruct((Q, D), jnp.float32)],
    )(feat2, Wm, bm, W1, b1, W2, b2, gamma, beta)[0]


def kernel(src, ref_point, src_query, Wq, bq, Wb, bb, Wk, bk, Woff, boff,
           Wa, ba, Wm, bm, W1, b1, W2, b2, gamma, beta):
    del Wq, bq
    srcq2 = src_query.reshape(Q, D)
    src2 = src.reshape(Q, D)
    rp3 = ref_point.reshape(B, P, 2)
    Woff2 = Woff.reshape(D, H, K, 2).transpose(0, 3, 1, 2).reshape(D, 2 * H * K)
    boff2 = boff.reshape(H, K, 2).transpose(2, 0, 1).reshape(2 * H * K)
    Wcat = jnp.concatenate([Woff2, Wa], axis=1)
    bcat = jnp.concatenate([boff2, ba]).reshape(1, -1)

    off, alog, tbl_t = _run_prep(srcq2, src2, Wb, bb.reshape(1, D), Wcat, bcat,
                                 Wk, bk.reshape(1, D))
    idx_t, w_t = _run_addr(off, alog, rp3)

    featflat = _run_sc(idx_t.reshape(NE), w_t.reshape(NE),
                       tbl_t.reshape(NITEMS, DK))

    featp = featflat.reshape(NB, 2, 8, 128)
    out = _run_post(featp, Wm, bm.reshape(1, D), W1, b1.reshape(1, DFF),
                    W2, b2.reshape(1, D), gamma.reshape(1, D), beta.reshape(1, D))
    return out.reshape(B, QH, QW, D)


# speedup vs baseline: 1.1258x; 1.0352x over previous
import functools

import jax
import jax.numpy as jnp
from jax import lax
from jax.experimental import pallas as pl
from jax.experimental.pallas import tpu as pltpu
from jax.experimental.pallas import tpu_sc as plsc

B, QH, QW = 2, 100, 100
D, H, K, SCALES = 256, 8, 8, 1
DK = D // H
DFF = 1024
P = QH * QW
Q = B * P
NITEMS = Q * H
NE = NITEMS * 32

_QB_A = 1000


NB = Q // 8


def _to_phys(x):
    r = x.shape[0]
    return x.reshape(r // 8, 8, 2, 128).transpose(0, 2, 1, 3)


def _from_phys(x):
    r = x.shape[0] * 8
    return x.transpose(0, 2, 1, 3).reshape(r, 256)


_RPB = _QB_A // QW


def _prep_body(srcq_ref, src_ref, Wb_ref, bb_ref, Wcat_ref, bcat_ref,
               Wk_ref, bk_ref, off_ref, alog_ref, tbl_ref, tbl_s):
    for j in range(_RPB):
        sq = jnp.dot(srcq_ref[j], Wb_ref[...], preferred_element_type=jnp.float32) + bb_ref[...]
        offa = jnp.dot(sq, Wcat_ref[...], preferred_element_type=jnp.float32) + bcat_ref[...]
        off_ref[pl.ds(j * QW, QW), :] = offa[:, :2 * H * K]
        alog_ref[pl.ds(j * QW, QW), :] = offa[:, 2 * H * K:]
        tbl_s[pl.ds(j * QW, QW), :] = (
            jnp.dot(src_ref[j], Wk_ref[...], preferred_element_type=jnp.float32) + bk_ref[...])
    tbl_ref[...] = _to_phys(tbl_s[...])


def _run_prep(srcq3, src3, Wb, bb, Wcat, bcat, Wk, bk):
    n = 2 * H * K + H * K
    return pl.pallas_call(
        _prep_body,
        grid=(Q // _QB_A,),
        in_specs=[
            pl.BlockSpec((_RPB, QW, D), lambda i: (i, 0, 0)),
            pl.BlockSpec((_RPB, QW, D), lambda i: (i, 0, 0)),
            pl.BlockSpec((D, D), lambda i: (0, 0)),
            pl.BlockSpec((1, D), lambda i: (0, 0)),
            pl.BlockSpec((D, n), lambda i: (0, 0)),
            pl.BlockSpec((1, n), lambda i: (0, 0)),
            pl.BlockSpec((D, D), lambda i: (0, 0)),
            pl.BlockSpec((1, D), lambda i: (0, 0)),
        ],
        out_specs=[
            pl.BlockSpec((_QB_A, 2 * H * K), lambda i: (i, 0)),
            pl.BlockSpec((_QB_A, H * K), lambda i: (i, 0)),
            pl.BlockSpec((_QB_A // 8, 2, 8, 128), lambda i: (i, 0, 0, 0)),
        ],
        out_shape=[
            jax.ShapeDtypeStruct((Q, 2 * H * K), jnp.float32),
            jax.ShapeDtypeStruct((Q, H * K), jnp.float32),
            jax.ShapeDtypeStruct((NB, 2, 8, 128), jnp.float32),
        ],
        scratch_shapes=[pltpu.VMEM((_QB_A, D), jnp.float32)],
    )(srcq3, src3, Wb, bb, Wcat, bcat, Wk, bk)


_QB_B = 2000

import numpy as _np

_LANE = _np.arange(256)
_LH = _LANE // 32
_LC = (_LANE % 32) // 8
_I64 = _np.arange(64)
_REP = (_I64[:, None] == (_LH * 8 + _LANE % 8)[None, :]).astype(_np.float32)
_RSUM = ((_I64 // 8)[:, None] == _LH[None, :]).astype(_np.float32)
_RX = _np.concatenate([_REP * (_LC % 2 == 0)[None, :], _REP * (_LC % 2 == 1)[None, :]], axis=0)
_RY = _np.concatenate([_REP * (_LC // 2 == 0)[None, :], _REP * (_LC // 2 == 1)[None, :]], axis=0)
_T2 = (_np.arange(2)[:, None] == ((_I64 // 8) % 2)[None, :]).astype(_np.float32)
_ROWC = ((_LH // 4) * 32 + _LH % 4).astype(_np.float32)


def _addr_body(off_ref, alog_ref, rp0_ref, rp1_ref, rep_ref, rsum_ref,
               rx_ref, ry_ref, t2_ref, idx_ref, w_ref):
    b = pl.program_id(0)
    off = off_ref[...]
    alog = alog_ref[...]
    rp0 = rp0_ref[0]
    rp1 = rp1_ref[0]
    offx = off[:, :64]
    offy = off[:, 64:]
    rpx2 = jnp.concatenate([rp0[:, 0:1], rp1[:, 0:1]], axis=1) * float(QW - 1)
    rpy2 = jnp.concatenate([rp0[:, 1:2], rp1[:, 1:2]], axis=1) * float(QH - 1)
    t2 = t2_ref[...]
    rrpx = jnp.dot(rpx2, t2, preferred_element_type=jnp.float32, precision=lax.Precision.HIGHEST)
    rrpy = jnp.dot(rpy2, t2, preferred_element_type=jnp.float32, precision=lax.Precision.HIGHEST)
    ptx = rrpx + offx
    pty = rrpy + offy
    vx = 2.0 * ptx / float(QW - 1) - 1.0
    vy = 2.0 * pty / float(QH - 1) - 1.0
    sx = ((vx + 1.0) * float(QW) - 1.0) / 2.0
    sy = ((vy + 1.0) * float(QH) - 1.0) / 2.0
    x0 = jnp.floor(sx)
    y0 = jnp.floor(sy)
    x1 = x0 + 1.0
    y1 = y0 + 1.0
    wx1 = sx - x0
    wx0 = 1.0 - wx1
    wy1 = sy - y0
    wy0 = 1.0 - wy1

    def fvalid(cf, lim):
        return ((cf >= 0.0) & (cf <= lim)).astype(jnp.float32)

    wvx = jnp.concatenate([wx0 * fvalid(x0, float(QW - 1)),
                           wx1 * fvalid(x1, float(QW - 1))], axis=1)
    wvy = jnp.concatenate([wy0 * fvalid(y0, float(QH - 1)),
                           wy1 * fvalid(y1, float(QH - 1))], axis=1)
    xi = jnp.concatenate([jnp.clip(x0, 0.0, float(QW - 1)),
                          jnp.clip(x1, 0.0, float(QW - 1))], axis=1)
    yi = jnp.concatenate([jnp.clip(y0, 0.0, float(QH - 1)),
                          jnp.clip(y1, 0.0, float(QH - 1))], axis=1)
    rx = rx_ref[...]
    ry = ry_ref[...]
    wvx256 = jnp.dot(wvx, rx, preferred_element_type=jnp.float32, precision=lax.Precision.HIGHEST)
    wvy256 = jnp.dot(wvy, ry, preferred_element_type=jnp.float32, precision=lax.Precision.HIGHEST)
    xi256 = jnp.dot(xi, rx, preferred_element_type=jnp.float32, precision=lax.Precision.HIGHEST)
    yi256 = jnp.dot(yi, ry, preferred_element_type=jnp.float32, precision=lax.Precision.HIGHEST)
    ea = jnp.exp(alog)
    a256 = jnp.dot(ea, rep_ref[...], preferred_element_type=jnp.float32, precision=lax.Precision.HIGHEST)
    s256 = jnp.dot(ea, rsum_ref[...], preferred_element_type=jnp.float32, precision=lax.Precision.HIGHEST)
    w256 = a256 / s256 * wvx256 * wvy256
    qs = (b * P).astype(jnp.float32) + yi256 * float(QW) + xi256
    qs8 = jnp.floor(qs * 0.125)
    hh = lax.broadcasted_iota(jnp.int32, (1, 256), 1) // 32
    rowc = ((hh // 4) * 32 + hh % 4).astype(jnp.float32)
    row = qs8 * 64.0 + (qs - qs8 * 8.0) * 4.0 + rowc
    idx_ref[...] = _to_phys(row.astype(jnp.int32))
    w_ref[...] = _to_phys(w256)


def _run_addr(off, alog, rp3):
    qbb = _QB_B
    return pl.pallas_call(
        _addr_body,
        grid=(B, P // qbb),
        in_specs=[
            pl.BlockSpec((qbb, 2 * H * K), lambda b, p: (b * (P // qbb) + p, 0)),
            pl.BlockSpec((qbb, H * K), lambda b, p: (b * (P // qbb) + p, 0)),
            pl.BlockSpec((1, qbb, 2), lambda b, p: (0, p, 0)),
            pl.BlockSpec((1, qbb, 2), lambda b, p: (1, p, 0)),
            pl.BlockSpec((64, 256), lambda b, p: (0, 0)),
            pl.BlockSpec((64, 256), lambda b, p: (0, 0)),
            pl.BlockSpec((128, 256), lambda b, p: (0, 0)),
            pl.BlockSpec((128, 256), lambda b, p: (0, 0)),
            pl.BlockSpec((2, 64), lambda b, p: (0, 0)),
        ],
        out_specs=[
            pl.BlockSpec((qbb // 8, 2, 8, 128), lambda b, p: (b * (P // qbb) + p, 0, 0, 0)),
            pl.BlockSpec((qbb // 8, 2, 8, 128), lambda b, p: (b * (P // qbb) + p, 0, 0, 0)),
        ],
        out_shape=[
            jax.ShapeDtypeStruct((NB, 2, 8, 128), jnp.int32),
            jax.ShapeDtypeStruct((NB, 2, 8, 128), jnp.float32),
        ],
    )(off, alog, rp3, rp3, _REP, _RSUM, _RX, _RY, _T2)


_NW = 32
_IPW = NITEMS // _NW
_CHUNK = 40
_NCHUNK = _IPW // _CHUNK
_CE = _CHUNK * 32
_GPC = _CE // 128


def _sc_body(idx_hbm, w_hbm, tbl_hbm, out_hbm, idx_v, w_v, rows_v, out_v,
             isem0, isem1, gsem0, gsem1, osem0, osem1):
    wid = lax.axis_index("s") * 2 + lax.axis_index("c")
    base_item = wid * _IPW
    isem = (isem0, isem1)
    gsem = (gsem0, gsem1)
    osem = (osem0, osem1)

    def e_of(c):
        return (base_item + c * _CHUNK) * 32

    def issue_idxw(c, q):
        e0 = e_of(c)
        pltpu.async_copy(idx_hbm.at[pl.ds(e0, _CE)], idx_v.at[q], isem[q])
        pltpu.async_copy(w_hbm.at[pl.ds(e0, _CE)], w_v.at[q], isem[q])

    def wait_idxw(q):
        pltpu.make_async_copy(idx_hbm.at[pl.ds(0, _CE)], idx_v.at[q], isem[q]).wait()
        pltpu.make_async_copy(w_hbm.at[pl.ds(0, _CE)], w_v.at[q], isem[q]).wait()

    def issue_gathers(q):
        for g in range(_GPC):
            pltpu.async_copy(tbl_hbm.at[idx_v.at[q, pl.ds(g * 128, 128)]],
                             rows_v.at[q, pl.ds(g * 128, 128)], gsem[q])

    def drain_gathers(q):
        pltpu.make_async_copy(tbl_hbm.at[pl.ds(0, _CE)], rows_v.at[q], gsem[q]).wait()

    def wait_out(q):
        pltpu.make_async_copy(out_v.at[q], out_hbm.at[pl.ds(0, _CE)], osem[q]).wait()

    def compute(c, q):
        def item_body(i, carry2):
            rbase = i * 32
            wvA = w_v[q, pl.ds(rbase, 16)]
            wvB = w_v[q, pl.ds(rbase + 16, 16)]
            a0 = jnp.zeros((16,), jnp.float32)
            a1 = jnp.zeros((16,), jnp.float32)
            a2 = jnp.zeros((16,), jnp.float32)
            a3 = jnp.zeros((16,), jnp.float32)
            for r in range(16):
                wgA = wvA[r]
                wgB = wvB[r]
                a0 = a0 + wgA * rows_v[q, rbase + r, pl.ds(0, 16)]
                a1 = a1 + wgA * rows_v[q, rbase + r, pl.ds(16, 16)]
                a2 = a2 + wgB * rows_v[q, rbase + 16 + r, pl.ds(0, 16)]
                a3 = a3 + wgB * rows_v[q, rbase + 16 + r, pl.ds(16, 16)]
            out_v[q, pl.ds(rbase, 16)] = a0 + a2
            out_v[q, pl.ds(rbase + 16, 16)] = a1 + a3
            return carry2

        lax.fori_loop(0, _CHUNK, item_body, 0)
        pltpu.async_copy(out_v.at[q], out_hbm.at[pl.ds(e_of(c), _CE)], osem[q])

    def step(c, q):
        @pl.when(c < _NCHUNK - 1)
        def _():
            wait_idxw(1 - q)
            issue_gathers(1 - q)
        drain_gathers(q)

        @pl.when(c >= 2)
        def _():
            wait_out(q)
        compute(c, q)

        @pl.when(c + 2 <= _NCHUNK - 1)
        def _():
            issue_idxw(c + 2, q)

    pltpu.sync_copy(idx_hbm.at[pl.ds(e_of(0), _CE)], idx_v.at[0])
    pltpu.sync_copy(w_hbm.at[pl.ds(e_of(0), _CE)], w_v.at[0])
    issue_gathers(0)
    issue_idxw(1, 1)

    def pair_body(j, carry):
        step(2 * j, 0)
        step(2 * j + 1, 1)
        return carry

    lax.fori_loop(0, _NCHUNK // 2, pair_body, 0)
    if _NCHUNK % 2:
        step(_NCHUNK - 1, 0)
    wait_out(0)
    wait_out(1)


def _run_sc(idx2, wflat, tbl):
    mesh = plsc.VectorSubcoreMesh(core_axis_name="c", subcore_axis_name="s")
    f = pl.kernel(
        _sc_body,
        out_type=jax.ShapeDtypeStruct((NE,), jnp.float32),
        mesh=mesh,
        compiler_params=pltpu.CompilerParams(use_tc_tiling_on_sc=False),
        scratch_types=[
            pltpu.VMEM((2, _CE), jnp.int32),
            pltpu.VMEM((2, _CE), jnp.float32),
            pltpu.VMEM((2, _CE, 32), jnp.float32),
            pltpu.VMEM((2, _CE), jnp.float32),
            pltpu.SemaphoreType.DMA,
            pltpu.SemaphoreType.DMA,
            pltpu.SemaphoreType.DMA,
            pltpu.SemaphoreType.DMA,
            pltpu.SemaphoreType.DMA,
            pltpu.SemaphoreType.DMA,
        ],
    )
    return f(idx2, wflat, tbl)


_QB_D = 1000


def _post_body(feat_ref, Wm_ref, bm_ref, W1_ref, b1_ref, W2_ref, b2_ref,
               g_ref, be_ref, out_ref):
    feat = _from_phys(feat_ref[...])
    x = jnp.dot(feat, Wm_ref[...], preferred_element_type=jnp.float32) + bm_ref[...]
    hh = jnp.maximum(jnp.dot(x, W1_ref[...], preferred_element_type=jnp.float32) + b1_ref[...], 0.0)
    y = jnp.dot(hh, W2_ref[...], preferred_element_type=jnp.float32) + b2_ref[...] + x
    mu = jnp.mean(y, axis=-1, keepdims=True)
    var = jnp.mean((y - mu) ** 2, axis=-1, keepdims=True)
    o = (y - mu) / jnp.sqrt(var + 1e-5) * g_ref[...] + be_ref[...]
    for j in range(_QB_D // QW):
        out_ref[j] = o[j * QW:(j + 1) * QW, :]


def _run_post(feat2, Wm, bm, W1, b1, W2, b2, gamma, beta):
    return pl.pallas_call(
        _post_body,
        grid=(Q // _QB_D,),
        in_specs=[
            pl.BlockSpec((_QB_D // 8, 2, 8, 128), lambda i: (i, 0, 0, 0)),
            pl.BlockSpec((D, D), lambda i: (0, 0)),
            pl.BlockSpec((1, D), lambda i: (0, 0)),
            pl.BlockSpec((D, DFF), lambda i: (0, 0)),
            pl.BlockSpec((1, DFF), lambda i: (0, 0)),
            pl.BlockSpec((DFF, D), lambda i: (0, 0)),
            pl.BlockSpec((1, D), lambda i: (0, 0)),
            pl.BlockSpec((1, D), lambda i: (0, 0)),
            pl.BlockSpec((1, D), lambda i: (0, 0)),
        ],
        out_specs=[pl.BlockSpec((_QB_D // QW, QW, D), lambda i: (i, 0, 0))],
        out_shape=[jax.ShapeDtypeStruct((B * QH, QW, D), jnp.float32)],
    )(feat2, Wm, bm, W1, b1, W2, b2, gamma, beta)[0]


def kernel(src, ref_point, src_query, Wq, bq, Wb, bb, Wk, bk, Woff, boff,
           Wa, ba, Wm, bm, W1, b1, W2, b2, gamma, beta):
    del Wq, bq
    srcq3 = src_query.reshape(B * QH, QW, D)
    src3 = src.reshape(B * QH, QW, D)
    rp3 = ref_point.reshape(B, P, 2)
    Woff2 = Woff.reshape(D, H, K, 2).transpose(0, 3, 1, 2).reshape(D, 2 * H * K)
    boff2 = boff.reshape(H, K, 2).transpose(2, 0, 1).reshape(2 * H * K)
    Wcat = jnp.concatenate([Woff2, Wa], axis=1)
    bcat = jnp.concatenate([boff2, ba]).reshape(1, -1)

    off, alog, tbl_t = _run_prep(srcq3, src3, Wb, bb.reshape(1, D), Wcat, bcat,
                                 Wk, bk.reshape(1, D))
    idx_t, w_t = _run_addr(off, alog, rp3)

    featflat = _run_sc(idx_t.reshape(NE), w_t.reshape(NE),
                       tbl_t.reshape(NITEMS, DK))

    featp = featflat.reshape(NB, 2, 8, 128)
    out = _run_post(featp, Wm, bm.reshape(1, D), W1, b1.reshape(1, DFF),
                    W2, b2.reshape(1, D), gamma.reshape(1, D), beta.reshape(1, D))
    return out.reshape(B, QH, QW, D)


# speedup vs baseline: 1.3469x; 1.1963x over previous
import functools

import jax
import jax.numpy as jnp
from jax import lax
from jax.experimental import pallas as pl
from jax.experimental.pallas import tpu as pltpu
from jax.experimental.pallas import tpu_sc as plsc

B, QH, QW = 2, 100, 100
D, H, K, SCALES = 256, 8, 8, 1
DK = D // H
DFF = 1024
P = QH * QW
Q = B * P
NITEMS = Q * H
NE = NITEMS * 32

_QB_A = 1000


NB = Q // 8


def _to_phys(x):
    r = x.shape[0]
    return x.reshape(r // 8, 8, 2, 128).transpose(0, 2, 1, 3)


def _from_phys(x):
    r = x.shape[0] * 8
    return x.transpose(0, 2, 1, 3).reshape(r, 256)


_RPB = _QB_A // QW


def _prep_body(srcq_ref, src_ref, Wb_ref, bb_ref, Wcat_ref, bcat_ref,
               Wk_ref, bk_ref, off_ref, alog_ref, tbl_ref, tbl_s):
    for j in range(_RPB):
        sq = jnp.dot(srcq_ref[0, j], Wb_ref[...], preferred_element_type=jnp.float32) + bb_ref[...]
        offa = jnp.dot(sq, Wcat_ref[...], preferred_element_type=jnp.float32) + bcat_ref[...]
        off_ref[pl.ds(j * QW, QW), :] = offa[:, :2 * H * K]
        alog_ref[pl.ds(j * QW, QW), :] = offa[:, 2 * H * K:]
        tbl_s[pl.ds(j * QW, QW), :] = (
            jnp.dot(src_ref[0, j], Wk_ref[...], preferred_element_type=jnp.float32) + bk_ref[...])
    tbl_ref[...] = _to_phys(tbl_s[...])


def _run_prep(srcq3, src3, Wb, bb, Wcat, bcat, Wk, bk):
    n = 2 * H * K + H * K
    return pl.pallas_call(
        _prep_body,
        grid=(Q // _QB_A,),
        in_specs=[
            pl.BlockSpec((1, _RPB, QW, D), lambda i: (i // (QH // _RPB), i % (QH // _RPB), 0, 0)),
            pl.BlockSpec((1, _RPB, QW, D), lambda i: (i // (QH // _RPB), i % (QH // _RPB), 0, 0)),
            pl.BlockSpec((D, D), lambda i: (0, 0)),
            pl.BlockSpec((1, D), lambda i: (0, 0)),
            pl.BlockSpec((D, n), lambda i: (0, 0)),
            pl.BlockSpec((1, n), lambda i: (0, 0)),
            pl.BlockSpec((D, D), lambda i: (0, 0)),
            pl.BlockSpec((1, D), lambda i: (0, 0)),
        ],
        out_specs=[
            pl.BlockSpec((_QB_A, 2 * H * K), lambda i: (i, 0)),
            pl.BlockSpec((_QB_A, H * K), lambda i: (i, 0)),
            pl.BlockSpec((_QB_A // 8, 2, 8, 128), lambda i: (i, 0, 0, 0)),
        ],
        out_shape=[
            jax.ShapeDtypeStruct((Q, 2 * H * K), jnp.float32),
            jax.ShapeDtypeStruct((Q, H * K), jnp.float32),
            jax.ShapeDtypeStruct((NB, 2, 8, 128), jnp.float32),
        ],
        scratch_shapes=[pltpu.VMEM((_QB_A, D), jnp.float32)],
    )(srcq3, src3, Wb, bb, Wcat, bcat, Wk, bk)


_QB_B = 2000

import numpy as _np

_LANE = _np.arange(256)
_LH = _LANE // 32
_LC = (_LANE % 32) // 8
_I64 = _np.arange(64)
_REP = (_I64[:, None] == (_LH * 8 + _LANE % 8)[None, :]).astype(_np.float32)
_RSUM = ((_I64 // 8)[:, None] == _LH[None, :]).astype(_np.float32)
_RX = _np.concatenate([_REP * (_LC % 2 == 0)[None, :], _REP * (_LC % 2 == 1)[None, :]], axis=0)
_RY = _np.concatenate([_REP * (_LC // 2 == 0)[None, :], _REP * (_LC // 2 == 1)[None, :]], axis=0)
_T2 = (_np.arange(2)[:, None] == ((_I64 // 8) % 2)[None, :]).astype(_np.float32)
_ROWC = ((_LH // 4) * 32 + _LH % 4).astype(_np.float32)


def _addr_body(off_ref, alog_ref, rp0_ref, rp1_ref, rep_ref, rsum_ref,
               rx_ref, ry_ref, t2_ref, idx_ref, w_ref):
    b = pl.program_id(0)
    off = off_ref[...]
    alog = alog_ref[...]
    rp0 = rp0_ref[0]
    rp1 = rp1_ref[0]
    offx = off[:, :64]
    offy = off[:, 64:]
    rpx2 = jnp.concatenate([rp0[:, 0:1], rp1[:, 0:1]], axis=1) * float(QW - 1)
    rpy2 = jnp.concatenate([rp0[:, 1:2], rp1[:, 1:2]], axis=1) * float(QH - 1)
    t2 = t2_ref[...]
    rrpx = jnp.dot(rpx2, t2, preferred_element_type=jnp.float32, precision=lax.Precision.HIGHEST)
    rrpy = jnp.dot(rpy2, t2, preferred_element_type=jnp.float32, precision=lax.Precision.HIGHEST)
    ptx = rrpx + offx
    pty = rrpy + offy
    vx = 2.0 * ptx / float(QW - 1) - 1.0
    vy = 2.0 * pty / float(QH - 1) - 1.0
    sx = ((vx + 1.0) * float(QW) - 1.0) / 2.0
    sy = ((vy + 1.0) * float(QH) - 1.0) / 2.0
    x0 = jnp.floor(sx)
    y0 = jnp.floor(sy)
    x1 = x0 + 1.0
    y1 = y0 + 1.0
    wx1 = sx - x0
    wx0 = 1.0 - wx1
    wy1 = sy - y0
    wy0 = 1.0 - wy1

    def fvalid(cf, lim):
        return ((cf >= 0.0) & (cf <= lim)).astype(jnp.float32)

    wvx = jnp.concatenate([wx0 * fvalid(x0, float(QW - 1)),
                           wx1 * fvalid(x1, float(QW - 1))], axis=1)
    wvy = jnp.concatenate([wy0 * fvalid(y0, float(QH - 1)),
                           wy1 * fvalid(y1, float(QH - 1))], axis=1)
    xi = jnp.concatenate([jnp.clip(x0, 0.0, float(QW - 1)),
                          jnp.clip(x1, 0.0, float(QW - 1))], axis=1)
    yi = jnp.concatenate([jnp.clip(y0, 0.0, float(QH - 1)),
                          jnp.clip(y1, 0.0, float(QH - 1))], axis=1)
    rx = rx_ref[...]
    ry = ry_ref[...]
    wvx256 = jnp.dot(wvx, rx, preferred_element_type=jnp.float32, precision=lax.Precision.HIGHEST)
    wvy256 = jnp.dot(wvy, ry, preferred_element_type=jnp.float32, precision=lax.Precision.HIGHEST)
    xi256 = jnp.dot(xi, rx, preferred_element_type=jnp.float32, precision=lax.Precision.HIGHEST)
    yi256 = jnp.dot(yi, ry, preferred_element_type=jnp.float32, precision=lax.Precision.HIGHEST)
    ea = jnp.exp(alog)
    a256 = jnp.dot(ea, rep_ref[...], preferred_element_type=jnp.float32, precision=lax.Precision.HIGHEST)
    s256 = jnp.dot(ea, rsum_ref[...], preferred_element_type=jnp.float32, precision=lax.Precision.HIGHEST)
    w256 = a256 / s256 * wvx256 * wvy256
    qs = (b * P).astype(jnp.float32) + yi256 * float(QW) + xi256
    qs8 = jnp.floor(qs * 0.125)
    hh = lax.broadcasted_iota(jnp.int32, (1, 256), 1) // 32
    rowc = ((hh // 4) * 32 + hh % 4).astype(jnp.float32)
    row = qs8 * 64.0 + (qs - qs8 * 8.0) * 4.0 + rowc
    idx_ref[...] = _to_phys(row.astype(jnp.int32))
    w_ref[...] = _to_phys(w256)


def _run_addr(off, alog, rp3):
    qbb = _QB_B
    return pl.pallas_call(
        _addr_body,
        grid=(B, P // qbb),
        in_specs=[
            pl.BlockSpec((qbb, 2 * H * K), lambda b, p: (b * (P // qbb) + p, 0)),
            pl.BlockSpec((qbb, H * K), lambda b, p: (b * (P // qbb) + p, 0)),
            pl.BlockSpec((1, qbb, 2), lambda b, p: (0, p, 0)),
            pl.BlockSpec((1, qbb, 2), lambda b, p: (1, p, 0)),
            pl.BlockSpec((64, 256), lambda b, p: (0, 0)),
            pl.BlockSpec((64, 256), lambda b, p: (0, 0)),
            pl.BlockSpec((128, 256), lambda b, p: (0, 0)),
            pl.BlockSpec((128, 256), lambda b, p: (0, 0)),
            pl.BlockSpec((2, 64), lambda b, p: (0, 0)),
        ],
        out_specs=[
            pl.BlockSpec((qbb // 8, 2, 8, 128), lambda b, p: (b * (P // qbb) + p, 0, 0, 0)),
            pl.BlockSpec((qbb // 8, 2, 8, 128), lambda b, p: (b * (P // qbb) + p, 0, 0, 0)),
        ],
        out_shape=[
            jax.ShapeDtypeStruct((NB, 2, 8, 128), jnp.int32),
            jax.ShapeDtypeStruct((NB, 2, 8, 128), jnp.float32),
        ],
    )(off, alog, rp3, rp3, _REP, _RSUM, _RX, _RY, _T2)


_NW = 32
_IPW = NITEMS // _NW
_CHUNK = 40
_NCHUNK = _IPW // _CHUNK
_CE = _CHUNK * 32
_GPC = _CE // 128


def _sc_body(idx_hbm, w_hbm, tbl_hbm, out_hbm, idx_v, w_v, rows_v, out_v,
             isem0, isem1, gsem0, gsem1, osem0, osem1):
    wid = lax.axis_index("s") * 2 + lax.axis_index("c")
    base_item = wid * _IPW
    isem = (isem0, isem1)
    gsem = (gsem0, gsem1)
    osem = (osem0, osem1)

    def e_of(c):
        return (base_item + c * _CHUNK) * 32

    def issue_idxw(c, q):
        e0 = e_of(c)
        pltpu.async_copy(idx_hbm.at[pl.ds(e0, _CE)], idx_v.at[q], isem[q])
        pltpu.async_copy(w_hbm.at[pl.ds(e0, _CE)], w_v.at[q], isem[q])

    def wait_idxw(q):
        pltpu.make_async_copy(idx_hbm.at[pl.ds(0, _CE)], idx_v.at[q], isem[q]).wait()
        pltpu.make_async_copy(w_hbm.at[pl.ds(0, _CE)], w_v.at[q], isem[q]).wait()

    def issue_gathers(q):
        for g in range(_GPC):
            pltpu.async_copy(tbl_hbm.at[idx_v.at[q, pl.ds(g * 128, 128)]],
                             rows_v.at[q, pl.ds(g * 128, 128)], gsem[q])

    def drain_gathers(q):
        pltpu.make_async_copy(tbl_hbm.at[pl.ds(0, _CE)], rows_v.at[q], gsem[q]).wait()

    def wait_out(q):
        pltpu.make_async_copy(out_v.at[q], out_hbm.at[pl.ds(0, _CE)], osem[q]).wait()

    def compute(c, q):
        def item_body(i, carry2):
            rbase = i * 32
            wvA = w_v[q, pl.ds(rbase, 16)]
            wvB = w_v[q, pl.ds(rbase + 16, 16)]
            a0 = jnp.zeros((16,), jnp.float32)
            a1 = jnp.zeros((16,), jnp.float32)
            a2 = jnp.zeros((16,), jnp.float32)
            a3 = jnp.zeros((16,), jnp.float32)
            for r in range(16):
                wgA = wvA[r]
                wgB = wvB[r]
                a0 = a0 + wgA * rows_v[q, rbase + r, pl.ds(0, 16)]
                a1 = a1 + wgA * rows_v[q, rbase + r, pl.ds(16, 16)]
                a2 = a2 + wgB * rows_v[q, rbase + 16 + r, pl.ds(0, 16)]
                a3 = a3 + wgB * rows_v[q, rbase + 16 + r, pl.ds(16, 16)]
            out_v[q, pl.ds(rbase, 16)] = a0 + a2
            out_v[q, pl.ds(rbase + 16, 16)] = a1 + a3
            return carry2

        lax.fori_loop(0, _CHUNK, item_body, 0)
        pltpu.async_copy(out_v.at[q], out_hbm.at[pl.ds(e_of(c), _CE)], osem[q])

    def step(c, q):
        @pl.when(c < _NCHUNK - 1)
        def _():
            wait_idxw(1 - q)
            issue_gathers(1 - q)
        drain_gathers(q)

        @pl.when(c >= 2)
        def _():
            wait_out(q)
        compute(c, q)

        @pl.when(c + 2 <= _NCHUNK - 1)
        def _():
            issue_idxw(c + 2, q)

    pltpu.sync_copy(idx_hbm.at[pl.ds(e_of(0), _CE)], idx_v.at[0])
    pltpu.sync_copy(w_hbm.at[pl.ds(e_of(0), _CE)], w_v.at[0])
    issue_gathers(0)
    issue_idxw(1, 1)

    def pair_body(j, carry):
        step(2 * j, 0)
        step(2 * j + 1, 1)
        return carry

    lax.fori_loop(0, _NCHUNK // 2, pair_body, 0)
    if _NCHUNK % 2:
        step(_NCHUNK - 1, 0)
    wait_out(0)
    wait_out(1)


def _run_sc(idx2, wflat, tbl):
    mesh = plsc.VectorSubcoreMesh(core_axis_name="c", subcore_axis_name="s")
    f = pl.kernel(
        _sc_body,
        out_type=jax.ShapeDtypeStruct((NE,), jnp.float32),
        mesh=mesh,
        compiler_params=pltpu.CompilerParams(use_tc_tiling_on_sc=False),
        scratch_types=[
            pltpu.VMEM((2, _CE), jnp.int32),
            pltpu.VMEM((2, _CE), jnp.float32),
            pltpu.VMEM((2, _CE, 32), jnp.float32),
            pltpu.VMEM((2, _CE), jnp.float32),
            pltpu.SemaphoreType.DMA,
            pltpu.SemaphoreType.DMA,
            pltpu.SemaphoreType.DMA,
            pltpu.SemaphoreType.DMA,
            pltpu.SemaphoreType.DMA,
            pltpu.SemaphoreType.DMA,
        ],
    )
    return f(idx2, wflat, tbl)


_QB_D = 1000


def _post_body(feat_ref, Wm_ref, bm_ref, W1_ref, b1_ref, W2_ref, b2_ref,
               g_ref, be_ref, out_ref):
    feat = _from_phys(feat_ref[...])
    x = jnp.dot(feat, Wm_ref[...], preferred_element_type=jnp.float32) + bm_ref[...]
    hh = jnp.maximum(jnp.dot(x, W1_ref[...], preferred_element_type=jnp.float32) + b1_ref[...], 0.0)
    y = jnp.dot(hh, W2_ref[...], preferred_element_type=jnp.float32) + b2_ref[...] + x
    mu = jnp.mean(y, axis=-1, keepdims=True)
    var = jnp.mean((y - mu) ** 2, axis=-1, keepdims=True)
    o = (y - mu) / jnp.sqrt(var + 1e-5) * g_ref[...] + be_ref[...]
    for j in range(_QB_D // QW):
        out_ref[0, j] = o[j * QW:(j + 1) * QW, :]


def _run_post(feat2, Wm, bm, W1, b1, W2, b2, gamma, beta):
    return pl.pallas_call(
        _post_body,
        grid=(Q // _QB_D,),
        in_specs=[
            pl.BlockSpec((_QB_D // 8, 2, 8, 128), lambda i: (i, 0, 0, 0)),
            pl.BlockSpec((D, D), lambda i: (0, 0)),
            pl.BlockSpec((1, D), lambda i: (0, 0)),
            pl.BlockSpec((D, DFF), lambda i: (0, 0)),
            pl.BlockSpec((1, DFF), lambda i: (0, 0)),
            pl.BlockSpec((DFF, D), lambda i: (0, 0)),
            pl.BlockSpec((1, D), lambda i: (0, 0)),
            pl.BlockSpec((1, D), lambda i: (0, 0)),
            pl.BlockSpec((1, D), lambda i: (0, 0)),
        ],
        out_specs=[pl.BlockSpec((1, _QB_D // QW, QW, D),
                                lambda i: (i // (QH // (_QB_D // QW)), i % (QH // (_QB_D // QW)), 0, 0))],
        out_shape=[jax.ShapeDtypeStruct((B, QH, QW, D), jnp.float32)],
    )(feat2, Wm, bm, W1, b1, W2, b2, gamma, beta)[0]


def kernel(src, ref_point, src_query, Wq, bq, Wb, bb, Wk, bk, Woff, boff,
           Wa, ba, Wm, bm, W1, b1, W2, b2, gamma, beta):
    del Wq, bq
    rp3 = ref_point.reshape(B, P, 2)
    Woff2 = Woff.reshape(D, H, K, 2).transpose(0, 3, 1, 2).reshape(D, 2 * H * K)
    boff2 = boff.reshape(H, K, 2).transpose(2, 0, 1).reshape(2 * H * K)
    Wcat = jnp.concatenate([Woff2, Wa], axis=1)
    bcat = jnp.concatenate([boff2, ba]).reshape(1, -1)

    off, alog, tbl_t = _run_prep(src_query, src, Wb, bb.reshape(1, D), Wcat, bcat,
                                 Wk, bk.reshape(1, D))
    idx_t, w_t = _run_addr(off, alog, rp3)

    featflat = _run_sc(idx_t.reshape(NE), w_t.reshape(NE),
                       tbl_t.reshape(NITEMS, DK))

    featp = featflat.reshape(NB, 2, 8, 128)
    return _run_post(featp, Wm, bm.reshape(1, D), W1, b1.reshape(1, DFF),
                     W2, b2.reshape(1, D), gamma.reshape(1, D), beta.reshape(1, D))
